# Initial kernel scaffold; baseline (speedup 1.0000x reference)
import jax
import jax.numpy as jnp
from jax.experimental import pallas as pl


def kernel(hidden, target, w0, b0, cluster_w, cluster_b, proj0, w1, b1, proj1, w2, b2, proj2):
    raise NotImplementedError("write your pallas kernel here")


# speedup vs baseline: 2.0235x; 2.0235x over previous
import functools

import jax
import jax.numpy as jnp
from jax.experimental import pallas as pl
from jax.experimental.pallas import tpu as pltpu

_C1 = 20000
_C2 = 60000


def _proj_kernel(x_ref, p_ref, o_ref):
    o_ref[...] = jnp.dot(x_ref[...], p_ref[...],
                         preferred_element_type=jnp.float32)


def _update(logits, eff, base, m_scr, l_scr, v_scr):
    T, V = logits.shape
    cols = base + jax.lax.broadcasted_iota(jnp.int32, (T, V), 1)
    m_prev = m_scr[...]
    m_new = jnp.maximum(m_prev, jnp.max(logits, axis=1, keepdims=True))
    l_scr[...] = (l_scr[...] * jnp.exp(m_prev - m_new)
                  + jnp.sum(jnp.exp(logits - m_new), axis=1, keepdims=True))
    m_scr[...] = m_new
    hit = cols == eff
    v_scr[...] += jnp.sum(jnp.where(hit, logits, 0.0), axis=1, keepdims=True)


def _head_flash(t_ref, x_ref, w_ref, b_ref, cw_ref, cb_ref,
                lse_ref, v_ref, m_scr, l_scr, v_scr, *, nsteps, vblk, vocab):
    j = pl.program_id(0)
    T = x_ref.shape[0]

    @pl.when(j == 0)
    def _init():
        m_scr[...] = jnp.full((T, 1), -jnp.inf, jnp.float32)
        l_scr[...] = jnp.zeros((T, 1), jnp.float32)
        v_scr[...] = jnp.zeros((T, 1), jnp.float32)

    x = x_ref[...]
    t = t_ref[...]
    logits = jax.lax.dot_general(x, w_ref[...], (((1,), (1,)), ((), ())),
                                 preferred_element_type=jnp.float32)
    logits = logits + b_ref[...]
    cols = j * vblk + jax.lax.broadcasted_iota(jnp.int32, (T, vblk), 1)
    logits = jnp.where(cols < vocab, logits, -1e30)
    eff = jnp.where(t < _C1, t, -1)
    _update(logits, eff, j * vblk, m_scr, l_scr, v_scr)

    @pl.when(j == nsteps - 1)
    def _cluster_and_out():
        clog = jax.lax.dot_general(x, cw_ref[...], (((1,), (1,)), ((), ())),
                                   preferred_element_type=jnp.float32)
        clog = clog + cb_ref[...]
        ccols = jax.lax.broadcasted_iota(jnp.int32, clog.shape, 1)
        clog = jnp.where(ccols < 2, clog, -1e30)
        ceff = jnp.where(t < _C1, -1, jnp.where(t < _C2, 1, 0))
        _update(clog, ceff, 0, m_scr, l_scr, v_scr)
        lse_ref[...] = m_scr[...] + jnp.log(l_scr[...])
        v_ref[...] = v_scr[...]


def _tail_flash(t_ref, x_ref, w_ref, b_ref,
                lse_ref, v_ref, m_scr, l_scr, v_scr,
                *, nsteps, vblk, vocab, left):
    j = pl.program_id(0)
    T = x_ref.shape[0]

    @pl.when(j == 0)
    def _init():
        m_scr[...] = jnp.full((T, 1), -jnp.inf, jnp.float32)
        l_scr[...] = jnp.zeros((T, 1), jnp.float32)
        v_scr[...] = jnp.zeros((T, 1), jnp.float32)

    x = x_ref[...]
    t = t_ref[...]
    logits = jax.lax.dot_general(x, w_ref[...], (((1,), (1,)), ((), ())),
                                 preferred_element_type=jnp.float32)
    logits = logits + b_ref[...]
    cols = j * vblk + jax.lax.broadcasted_iota(jnp.int32, (T, vblk), 1)
    logits = jnp.where(cols < vocab, logits, -1e30)
    eff = jnp.clip(t - left, 0, vocab - 1)
    _update(logits, eff, j * vblk, m_scr, l_scr, v_scr)

    @pl.when(j == nsteps - 1)
    def _out():
        lse_ref[...] = m_scr[...] + jnp.log(l_scr[...])
        v_ref[...] = v_scr[...]


def _combine(t_ref, lh_ref, vh_ref, l1_ref, v1_ref, l2_ref, v2_ref, o_ref):
    t = t_ref[...]
    nll = lh_ref[...] - vh_ref[...]
    in1 = (t >= _C1) & (t < _C2)
    in2 = t >= _C2
    nll = nll + jnp.where(in1, l1_ref[...] - v1_ref[...], 0.0)
    nll = nll + jnp.where(in2, l2_ref[...] - v2_ref[...], 0.0)
    o_ref[...] = nll


def _flash_call(body, t2, xp, w, b, extra=(), extra_specs=(), *, vblk, **kw):
    T = t2.shape[0]
    vocab = w.shape[0]
    K = w.shape[1]
    nsteps = pl.cdiv(vocab, vblk)
    return pl.pallas_call(
        functools.partial(body, nsteps=nsteps, vblk=vblk, vocab=vocab, **kw),
        grid=(nsteps,),
        in_specs=[
            pl.BlockSpec((T, 1), lambda j: (0, 0)),
            pl.BlockSpec((T, K), lambda j: (0, 0)),
            pl.BlockSpec((vblk, K), lambda j: (j, 0)),
            pl.BlockSpec((1, vblk), lambda j: (0, j)),
            *extra_specs,
        ],
        out_specs=[pl.BlockSpec((T, 1), lambda j: (0, 0))] * 2,
        out_shape=[jax.ShapeDtypeStruct((T, 1), jnp.float32)] * 2,
        scratch_shapes=[pltpu.VMEM((T, 1), jnp.float32)] * 3,
    )(t2, xp, w, b.reshape(1, -1), *extra)


def kernel(hidden, target, w0, b0, cluster_w, cluster_b, proj0,
           w1, b1, proj1, w2, b2, proj2):
    B, S, K = hidden.shape
    T = B * S
    k0 = proj0.shape[1]
    k1 = proj1.shape[1]
    k2 = proj2.shape[1]
    h2 = hidden.reshape(T, K)
    t2 = target.reshape(T, 1).astype(jnp.int32)

    P = jnp.concatenate([proj0, proj1, proj2], axis=1)
    npad = (-P.shape[1]) % 128
    P = jnp.pad(P, ((0, 0), (0, npad)))
    xp = pl.pallas_call(
        _proj_kernel,
        out_shape=jax.ShapeDtypeStruct((T, P.shape[1]), jnp.float32),
    )(h2, P)
    xp0 = xp[:, :k0]
    xp1 = xp[:, k0:k0 + k1]
    xp2 = xp[:, k0 + k1:k0 + k1 + k2]

    cwp = jnp.pad(cluster_w, ((0, 128 - cluster_w.shape[0]), (0, 0)))
    cbp = jnp.pad(cluster_b.reshape(1, -1), ((0, 0), (0, 128 - cluster_b.shape[0])))

    lse_h, v_h = _flash_call(
        _head_flash, t2, xp0, w0, b0,
        extra=(cwp, cbp),
        extra_specs=(pl.BlockSpec((128, k0), lambda j: (0, 0)),
                     pl.BlockSpec((1, 128), lambda j: (0, 0))),
        vblk=1024)
    lse_1, v_1 = _flash_call(_tail_flash, t2, xp1, w1, b1, vblk=2048, left=_C1)
    lse_2, v_2 = _flash_call(_tail_flash, t2, xp2, w2, b2, vblk=2048, left=_C2)

    nll = pl.pallas_call(
        _combine,
        out_shape=jax.ShapeDtypeStruct((T, 1), jnp.float32),
    )(t2, lse_h, v_h, lse_1, v_1, lse_2, v_2)
    return nll.reshape(target.shape)


# speedup vs baseline: 2.2572x; 1.1155x over previous
import functools

import jax
import jax.numpy as jnp
from jax.experimental import pallas as pl
from jax.experimental.pallas import tpu as pltpu

_C1 = 20000
_C2 = 60000
_NEG = -1e30


def _proj_kernel(x_ref, p_ref, o_ref):
    o_ref[...] = jnp.dot(x_ref[...].astype(jnp.bfloat16),
                         p_ref[...].astype(jnp.bfloat16),
                         preferred_element_type=jnp.float32).astype(jnp.bfloat16)


def _update(logits, eff, base, m_scr, l_scr, v_scr):
    T, V = logits.shape
    cols = base + jax.lax.broadcasted_iota(jnp.int32, (T, V), 1)
    m_prev = m_scr[...]
    m_new = jnp.maximum(m_prev, jnp.max(logits, axis=1, keepdims=True))
    l_scr[...] = (l_scr[...] * jnp.exp(m_prev - m_new)
                  + jnp.sum(jnp.exp(logits - m_new), axis=1, keepdims=True))
    m_scr[...] = m_new
    hit = cols == eff
    v_scr[...] += jnp.sum(jnp.where(hit, logits, 0.0), axis=1, keepdims=True)


def _masked_w_bf16(w_ref, j, vblk, vocab):
    rows = j * vblk + jax.lax.broadcasted_iota(jnp.int32, (vblk, 1), 0)
    return jnp.where(rows < vocab, w_ref[...], 0.0).astype(jnp.bfloat16)


def _head_flash(t_ref, x_ref, w_ref, b_ref, cw_ref, cb_ref,
                lse_ref, v_ref, m_scr, l_scr, v_scr, *, nsteps, vblk, vocab):
    j = pl.program_id(0)
    T = x_ref.shape[0]

    @pl.when(j == 0)
    def _init():
        m_scr[...] = jnp.full((T, 1), -jnp.inf, jnp.float32)
        l_scr[...] = jnp.zeros((T, 1), jnp.float32)
        v_scr[...] = jnp.zeros((T, 1), jnp.float32)

    x = x_ref[...]
    t = t_ref[...]
    logits = jax.lax.dot_general(x, _masked_w_bf16(w_ref, j, vblk, vocab),
                                 (((1,), (1,)), ((), ())),
                                 preferred_element_type=jnp.float32)
    logits = logits + b_ref[...]
    eff = jnp.where(t < _C1, t, -1)
    _update(logits, eff, j * vblk, m_scr, l_scr, v_scr)

    @pl.when(j == nsteps - 1)
    def _cluster_and_out():
        clog = jax.lax.dot_general(x, cw_ref[...].astype(jnp.bfloat16),
                                   (((1,), (1,)), ((), ())),
                                   preferred_element_type=jnp.float32)
        clog = clog + cb_ref[...]
        ceff = jnp.where(t < _C1, -1, jnp.where(t < _C2, 1, 0))
        _update(clog, ceff, 0, m_scr, l_scr, v_scr)
        lse_ref[...] = m_scr[...] + jnp.log(l_scr[...])
        v_ref[...] = v_scr[...]


def _tail_flash(t_ref, x_ref, w_ref, b_ref,
                lse_ref, v_ref, m_scr, l_scr, v_scr,
                *, nsteps, vblk, vocab, left):
    j = pl.program_id(0)
    T = x_ref.shape[0]

    @pl.when(j == 0)
    def _init():
        m_scr[...] = jnp.full((T, 1), -jnp.inf, jnp.float32)
        l_scr[...] = jnp.zeros((T, 1), jnp.float32)
        v_scr[...] = jnp.zeros((T, 1), jnp.float32)

    x = x_ref[...]
    t = t_ref[...]
    logits = jax.lax.dot_general(x, _masked_w_bf16(w_ref, j, vblk, vocab),
                                 (((1,), (1,)), ((), ())),
                                 preferred_element_type=jnp.float32)
    logits = logits + b_ref[...]
    eff = jnp.clip(t - left, 0, vocab - 1)
    _update(logits, eff, j * vblk, m_scr, l_scr, v_scr)

    @pl.when(j == nsteps - 1)
    def _out():
        lse_ref[...] = m_scr[...] + jnp.log(l_scr[...])
        v_ref[...] = v_scr[...]


def _combine(t_ref, lh_ref, vh_ref, l1_ref, v1_ref, l2_ref, v2_ref, o_ref):
    t = t_ref[...]
    nll = lh_ref[...] - vh_ref[...]
    in1 = (t >= _C1) & (t < _C2)
    in2 = t >= _C2
    nll = nll + jnp.where(in1, l1_ref[...] - v1_ref[...], 0.0)
    nll = nll + jnp.where(in2, l2_ref[...] - v2_ref[...], 0.0)
    o_ref[...] = nll


def _flash_call(body, t2, xp, w, b, extra=(), extra_specs=(), *, vblk, **kw):
    T = t2.shape[0]
    vocab = w.shape[0]
    K = w.shape[1]
    nsteps = pl.cdiv(vocab, vblk)
    bp = jnp.pad(b.reshape(1, -1), ((0, 0), (0, nsteps * vblk - vocab)),
                 constant_values=_NEG)
    return pl.pallas_call(
        functools.partial(body, nsteps=nsteps, vblk=vblk, vocab=vocab, **kw),
        grid=(nsteps,),
        in_specs=[
            pl.BlockSpec((T, 1), lambda j: (0, 0)),
            pl.BlockSpec((T, K), lambda j: (0, 0)),
            pl.BlockSpec((vblk, K), lambda j: (j, 0)),
            pl.BlockSpec((1, vblk), lambda j: (0, j)),
            *extra_specs,
        ],
        out_specs=[pl.BlockSpec((T, 1), lambda j: (0, 0))] * 2,
        out_shape=[jax.ShapeDtypeStruct((T, 1), jnp.float32)] * 2,
        scratch_shapes=[pltpu.VMEM((T, 1), jnp.float32)] * 3,
    )(t2, xp, w, bp, *extra)


def kernel(hidden, target, w0, b0, cluster_w, cluster_b, proj0,
           w1, b1, proj1, w2, b2, proj2):
    B, S, K = hidden.shape
    T = B * S
    k0 = proj0.shape[1]
    k1 = proj1.shape[1]
    k2 = proj2.shape[1]
    h2 = hidden.reshape(T, K)
    t2 = target.reshape(T, 1).astype(jnp.int32)

    P = jnp.concatenate([proj0, proj1, proj2], axis=1)
    npad = (-P.shape[1]) % 128
    P = jnp.pad(P, ((0, 0), (0, npad)))
    xp = pl.pallas_call(
        _proj_kernel,
        out_shape=jax.ShapeDtypeStruct((T, P.shape[1]), jnp.bfloat16),
    )(h2, P)
    xp0 = xp[:, :k0]
    xp1 = xp[:, k0:k0 + k1]
    xp2 = xp[:, k0 + k1:k0 + k1 + k2]

    cwp = jnp.pad(cluster_w, ((0, 128 - cluster_w.shape[0]), (0, 0)))
    cbp = jnp.pad(cluster_b.reshape(1, -1),
                  ((0, 0), (0, 128 - cluster_b.shape[0])),
                  constant_values=_NEG)

    lse_h, v_h = _flash_call(
        _head_flash, t2, xp0, w0, b0,
        extra=(cwp, cbp),
        extra_specs=(pl.BlockSpec((128, k0), lambda j: (0, 0)),
                     pl.BlockSpec((1, 128), lambda j: (0, 0))),
        vblk=2048)
    lse_1, v_1 = _flash_call(_tail_flash, t2, xp1, w1, b1, vblk=2048, left=_C1)
    lse_2, v_2 = _flash_call(_tail_flash, t2, xp2, w2, b2, vblk=2048, left=_C2)

    nll = pl.pallas_call(
        _combine,
        out_shape=jax.ShapeDtypeStruct((T, 1), jnp.float32),
    )(t2, lse_h, v_h, lse_1, v_1, lse_2, v_2)
    return nll.reshape(target.shape)


# speedup vs baseline: 2.2937x; 1.0162x over previous
import functools

import jax
import jax.numpy as jnp
from jax.experimental import pallas as pl
from jax.experimental.pallas import tpu as pltpu

_C1 = 20000
_C2 = 60000
_NEG = -1e30


def _proj_kernel(x_ref, p_ref, o_ref):
    o_ref[...] = jnp.dot(x_ref[...].astype(jnp.bfloat16),
                         p_ref[...].astype(jnp.bfloat16),
                         preferred_element_type=jnp.float32).astype(jnp.bfloat16)


def _flash(t_ref, x_ref, w_ref, b_ref, m_ref, s_ref, v_ref,
           *, vblk, vocab, left, shortlist):
    j = pl.program_id(0)
    rows = jax.lax.broadcasted_iota(jnp.int32, (vblk, 1), 0)
    w = jnp.where(j * vblk + rows < vocab, w_ref[...], 0.0).astype(jnp.bfloat16)
    logits = jax.lax.dot_general(w, x_ref[...], (((1,), (1,)), ((), ())),
                                 preferred_element_type=jnp.float32)
    logits = logits + b_ref[...]
    t = t_ref[...]
    if shortlist:
        eff = jnp.where(t < _C1, t, -1)
    else:
        eff = jnp.clip(t - left, 0, vocab - 1)
    eff = eff - j * vblk
    m = jnp.max(logits, axis=0, keepdims=True)
    s = jnp.sum(jnp.exp(logits - m), axis=0, keepdims=True)
    hit = rows == eff
    v = jnp.sum(jnp.where(hit, logits, 0.0), axis=0, keepdims=True)
    m_ref[...] = m[None]
    s_ref[...] = s[None]
    v_ref[...] = v[None]


def _combine(t_ref, x_ref, cw_ref, cb_ref,
             mh_ref, sh_ref, vh_ref, m1_ref, s1_ref, v1_ref,
             m2_ref, s2_ref, v2_ref, o_ref):
    t = t_ref[...]

    def lse_v(m_ref, s_ref, v_ref, extra_m=None, extra_s=None, extra_v=None):
        m = m_ref[:, 0, :]
        M = jnp.max(m, axis=0, keepdims=True)
        if extra_m is not None:
            M = jnp.maximum(M, extra_m)
        ssum = jnp.sum(s_ref[:, 0, :] * jnp.exp(m - M), axis=0, keepdims=True)
        if extra_s is not None:
            ssum = ssum + extra_s * jnp.exp(extra_m - M)
        v = jnp.sum(v_ref[:, 0, :], axis=0, keepdims=True)
        if extra_v is not None:
            v = v + extra_v
        return M + jnp.log(ssum), v

    clog = jax.lax.dot_general(cw_ref[...].astype(jnp.bfloat16), x_ref[...],
                               (((1,), (1,)), ((), ())),
                               preferred_element_type=jnp.float32)
    clog = clog + cb_ref[...]
    crows = jax.lax.broadcasted_iota(jnp.int32, clog.shape, 0)
    ceff = jnp.where(t < _C1, -1, jnp.where(t < _C2, 1, 0))
    cm = jnp.max(clog, axis=0, keepdims=True)
    cs = jnp.sum(jnp.exp(clog - cm), axis=0, keepdims=True)
    cv = jnp.sum(jnp.where(crows == ceff, clog, 0.0), axis=0, keepdims=True)

    lse_h, v_h = lse_v(mh_ref, sh_ref, vh_ref, cm, cs, cv)
    lse_1, v_1 = lse_v(m1_ref, s1_ref, v1_ref)
    lse_2, v_2 = lse_v(m2_ref, s2_ref, v2_ref)

    nll = lse_h - v_h
    in1 = (t >= _C1) & (t < _C2)
    in2 = t >= _C2
    nll = nll + jnp.where(in1, lse_1 - v_1, 0.0)
    nll = nll + jnp.where(in2, lse_2 - v_2, 0.0)
    o_ref[...] = nll


def _flash_call(t1, xp, w, b, *, vblk, left, shortlist):
    T = t1.shape[1]
    vocab, K = w.shape
    nsteps = pl.cdiv(vocab, vblk)
    bp = jnp.pad(b.reshape(-1, 1), ((0, nsteps * vblk - vocab), (0, 0)),
                 constant_values=_NEG)
    return pl.pallas_call(
        functools.partial(_flash, vblk=vblk, vocab=vocab, left=left,
                          shortlist=shortlist),
        grid=(nsteps,),
        in_specs=[
            pl.BlockSpec((1, T), lambda j: (0, 0)),
            pl.BlockSpec((T, K), lambda j: (0, 0)),
            pl.BlockSpec((vblk, K), lambda j: (j, 0)),
            pl.BlockSpec((vblk, 1), lambda j: (j, 0)),
        ],
        out_specs=[pl.BlockSpec((1, 1, T), lambda j: (j, 0, 0))] * 3,
        out_shape=[jax.ShapeDtypeStruct((nsteps, 1, T), jnp.float32)] * 3,
    )(t1, xp, w, bp)


def kernel(hidden, target, w0, b0, cluster_w, cluster_b, proj0,
           w1, b1, proj1, w2, b2, proj2):
    B, S, K = hidden.shape
    T = B * S
    k0 = proj0.shape[1]
    k1 = proj1.shape[1]
    k2 = proj2.shape[1]
    h2 = hidden.reshape(T, K)
    t1 = target.reshape(1, T).astype(jnp.int32)

    P = jnp.concatenate([proj0, proj1, proj2], axis=1)
    npad = (-P.shape[1]) % 128
    P = jnp.pad(P, ((0, 0), (0, npad)))
    xp = pl.pallas_call(
        _proj_kernel,
        out_shape=jax.ShapeDtypeStruct((T, P.shape[1]), jnp.bfloat16),
    )(h2, P)
    xp0 = xp[:, :k0]
    xp1 = xp[:, k0:k0 + k1]
    xp2 = xp[:, k0 + k1:k0 + k1 + k2]

    mh, sh, vh = _flash_call(t1, xp0, w0, b0, vblk=1024, left=0,
                             shortlist=True)
    m1, s1, v1 = _flash_call(t1, xp1, w1, b1, vblk=2048, left=_C1,
                             shortlist=False)
    m2, s2, v2 = _flash_call(t1, xp2, w2, b2, vblk=2048, left=_C2,
                             shortlist=False)

    cwp = jnp.pad(cluster_w, ((0, 8 - cluster_w.shape[0]), (0, 0)))
    cbp = jnp.pad(cluster_b.reshape(-1, 1),
                  ((0, 8 - cluster_b.shape[0]), (0, 0)),
                  constant_values=_NEG)

    nll = pl.pallas_call(
        _combine,
        out_shape=jax.ShapeDtypeStruct((1, T), jnp.float32),
    )(t1, xp0, cwp, cbp, mh, sh, vh, m1, s1, v1, m2, s2, v2)
    return nll.reshape(target.shape)
